# Initial kernel scaffold; baseline (speedup 1.0000x reference)
import jax
import jax.numpy as jnp
from jax.experimental import pallas as pl


def kernel(x, sel_dst, data_to_q, data_to_kv, out_proj, pos_to_pk, scale, pos_encoding):
    raise NotImplementedError("write your pallas kernel here")


# speedup vs baseline: 82.8173x; 82.8173x over previous
import functools
import math

import jax
import jax.numpy as jnp
from jax.experimental import pallas as pl
from jax.experimental.pallas import tpu as pltpu


def _matmul_kernel(x_ref, w_ref, o_ref):
    o_ref[...] = jnp.dot(x_ref[...], w_ref[...],
                         preferred_element_type=jnp.float32)


def _attn_kernel(q_ref, sel_ref, k_ref, v_ref, pk_ref, wo_ref, scale_ref,
                 o_ref, *, nq, bq, seq, e, p):
    i = pl.program_id(1)
    band0 = (nq - 1 - i) * bq
    w = seq + bq
    kmat = k_ref[0]
    vmat = v_ref[0]
    pband = pk_ref[pl.ds(band0, w), :]
    gates = jax.nn.sigmoid(sel_ref[0])
    sc = scale_ref[0, 0]
    outs = []
    for ei in range(e):
        q = q_ref[0, :, ei * p:(ei + 1) * p]
        pb = jax.lax.dot_general(q, pband, (((1,), (1,)), ((), ())),
                                 preferred_element_type=jnp.float32)
        posm = pltpu.roll(pb, w - (bq - 1), 1, stride=1, stride_axis=0)
        scores = jax.lax.dot_general(q, kmat, (((1,), (1,)), ((), ())),
                                     preferred_element_type=jnp.float32)
        scores = (scores + posm[:, :seq]) * sc
        m = jnp.max(scores, axis=-1, keepdims=True)
        ex = jnp.exp(scores - m)
        att = ex / jnp.sum(ex, axis=-1, keepdims=True)
        out_e = jnp.dot(att, vmat, preferred_element_type=jnp.float32)
        outs.append(out_e * gates[:, ei:ei + 1])
    acc = jnp.concatenate(outs, axis=1)
    o_ref[0] = jnp.dot(acc, wo_ref[...],
                       preferred_element_type=jnp.float32)


def kernel(x, sel_dst, data_to_q, data_to_kv, out_proj, pos_to_pk, scale,
           pos_encoding):
    B, S, D = x.shape
    E, _, P = data_to_q.shape
    EP = E * P
    L = pos_encoding.shape[0]

    wq = data_to_q.transpose(1, 0, 2).reshape(D, EP)
    ncols = EP + 2 * P + E
    ncols_pad = ((ncols + 127) // 128) * 128
    w_all = jnp.concatenate(
        [wq, data_to_kv, sel_dst.T,
         jnp.zeros((D, ncols_pad - ncols), jnp.float32)], axis=1)
    xf = x.reshape(B * S, D)
    rb = min(512, B * S)
    proj = pl.pallas_call(
        _matmul_kernel,
        grid=(B * S // rb,),
        in_specs=[pl.BlockSpec((rb, D), lambda r: (r, 0)),
                  pl.BlockSpec((D, ncols_pad), lambda r: (0, 0))],
        out_specs=pl.BlockSpec((rb, ncols_pad), lambda r: (r, 0)),
        out_shape=jax.ShapeDtypeStruct((B * S, ncols_pad), jnp.float32),
    )(xf, w_all)
    q_all = proj[:, :EP].reshape(B, S, EP)
    kk = proj[:, EP:EP + P].reshape(B, S, P)
    vv = proj[:, EP + P:EP + 2 * P].reshape(B, S, P)
    sel = proj[:, EP + 2 * P:EP + 2 * P + E].reshape(B, S, E)

    pe_pad = jnp.concatenate(
        [pos_encoding, jnp.zeros((2 * S - L, D), jnp.float32)], axis=0)
    prb = min(2048, 2 * S)
    pos_k = pl.pallas_call(
        _matmul_kernel,
        grid=(2 * S // prb,),
        in_specs=[pl.BlockSpec((prb, D), lambda r: (r, 0)),
                  pl.BlockSpec((D, P), lambda r: (0, 0))],
        out_specs=pl.BlockSpec((prb, P), lambda r: (r, 0)),
        out_shape=jax.ShapeDtypeStruct((2 * S, P), jnp.float32),
    )(pe_pad, pos_to_pk.T)

    bq = min(256, S)
    nq = S // bq
    wo = out_proj.reshape(EP, D)
    scale2 = scale.reshape(1, 1)
    out = pl.pallas_call(
        functools.partial(_attn_kernel, nq=nq, bq=bq, seq=S, e=E, p=P),
        grid=(B, nq),
        in_specs=[
            pl.BlockSpec((1, bq, EP), lambda b, i: (b, i, 0)),
            pl.BlockSpec((1, bq, E), lambda b, i: (b, i, 0)),
            pl.BlockSpec((1, S, P), lambda b, i: (b, 0, 0)),
            pl.BlockSpec((1, S, P), lambda b, i: (b, 0, 0)),
            pl.BlockSpec((2 * S, P), lambda b, i: (0, 0)),
            pl.BlockSpec((EP, D), lambda b, i: (0, 0)),
            pl.BlockSpec((1, 1), lambda b, i: (0, 0)),
        ],
        out_specs=pl.BlockSpec((1, bq, D), lambda b, i: (b, i, 0)),
        out_shape=jax.ShapeDtypeStruct((B, S, D), jnp.float32),
        compiler_params=pltpu.CompilerParams(
            dimension_semantics=("parallel", "parallel")),
    )(q_all, sel, kk, vv, pos_k, wo, scale2)
    return out


# speedup vs baseline: 123.7275x; 1.4940x over previous
import functools
import math

import jax
import jax.numpy as jnp
from jax.experimental import pallas as pl
from jax.experimental.pallas import tpu as pltpu

_LOG2E = 1.4426950408889634


def _proj_kernel(x_ref, w_ref, q_ref, k_ref, v_ref, sel_ref, *, ep, p, e):
    y = jnp.dot(x_ref[...].astype(jnp.bfloat16), w_ref[...],
                preferred_element_type=jnp.float32)
    q_ref[...] = y[:, :ep].astype(jnp.bfloat16)
    k_ref[...] = y[:, ep:ep + p].astype(jnp.bfloat16)
    v_ref[...] = y[:, ep + p:ep + 2 * p].astype(jnp.bfloat16)
    sel_ref[...] = y[:, ep + 2 * p:ep + 2 * p + e]


def _posk_kernel(x_ref, w_ref, o_ref):
    o_ref[...] = jnp.dot(x_ref[...].astype(jnp.bfloat16), w_ref[...],
                         preferred_element_type=jnp.float32
                         ).astype(jnp.bfloat16)


def _attn_kernel(q_ref, sel_ref, k_ref, v_ref, pk_ref, wo_ref,
                 o_ref, *, nq, bq, seq, e, p):
    i = pl.program_id(1)
    band0 = (nq - 1 - i) * bq
    w = seq + bq
    kmat = k_ref[0]
    vmat = v_ref[0]
    pband = pk_ref[pl.ds(band0, w), :]
    gates = jax.nn.sigmoid(sel_ref[0])
    outs = []
    for ei in range(e):
        q = q_ref[0, :, ei * p:(ei + 1) * p]
        pb = jax.lax.dot_general(q, pband, (((1,), (1,)), ((), ())),
                                 preferred_element_type=jnp.float32)
        posm = pltpu.roll(pb, w - (bq - 1), 1, stride=1, stride_axis=0)
        scores = jax.lax.dot_general(q, kmat, (((1,), (1,)), ((), ())),
                                     preferred_element_type=jnp.float32)
        ex = jnp.exp2(scores + posm[:, :seq])
        ssum = jnp.sum(ex, axis=-1, keepdims=True)
        out_e = jnp.dot(ex.astype(jnp.bfloat16), vmat,
                        preferred_element_type=jnp.float32)
        outs.append((out_e * (gates[:, ei:ei + 1] / ssum))
                    .astype(jnp.bfloat16))
    acc = jnp.concatenate(outs, axis=1)
    o_ref[0] = jnp.dot(acc, wo_ref[...],
                       preferred_element_type=jnp.float32)


def kernel(x, sel_dst, data_to_q, data_to_kv, out_proj, pos_to_pk, scale,
           pos_encoding):
    B, S, D = x.shape
    E, _, P = data_to_q.shape
    EP = E * P
    L = pos_encoding.shape[0]

    qscale = scale[0] * _LOG2E
    wq = data_to_q.transpose(1, 0, 2).reshape(D, EP) * qscale
    ncols = EP + 2 * P + E
    ncols_pad = ((ncols + 127) // 128) * 128
    w_all = jnp.concatenate(
        [wq, data_to_kv, sel_dst.T,
         jnp.zeros((D, ncols_pad - ncols), jnp.float32)],
        axis=1).astype(jnp.bfloat16)
    xf = x.reshape(B * S, D)
    rb = min(512, B * S)
    q_all, kk, vv, sel = pl.pallas_call(
        functools.partial(_proj_kernel, ep=EP, p=P, e=E),
        grid=(B * S // rb,),
        in_specs=[pl.BlockSpec((rb, D), lambda r: (r, 0)),
                  pl.BlockSpec((D, ncols_pad), lambda r: (0, 0))],
        out_specs=[pl.BlockSpec((rb, EP), lambda r: (r, 0)),
                   pl.BlockSpec((rb, P), lambda r: (r, 0)),
                   pl.BlockSpec((rb, P), lambda r: (r, 0)),
                   pl.BlockSpec((rb, E), lambda r: (r, 0))],
        out_shape=[jax.ShapeDtypeStruct((B * S, EP), jnp.bfloat16),
                   jax.ShapeDtypeStruct((B * S, P), jnp.bfloat16),
                   jax.ShapeDtypeStruct((B * S, P), jnp.bfloat16),
                   jax.ShapeDtypeStruct((B * S, E), jnp.float32)],
    )(xf, w_all)
    q_all = q_all.reshape(B, S, EP)
    kk = kk.reshape(B, S, P)
    vv = vv.reshape(B, S, P)
    sel = sel.reshape(B, S, E)

    pe_pad = jnp.concatenate(
        [pos_encoding, jnp.zeros((2 * S - L, D), jnp.float32)], axis=0)
    prb = min(2048, 2 * S)
    pos_k = pl.pallas_call(
        _posk_kernel,
        grid=(2 * S // prb,),
        in_specs=[pl.BlockSpec((prb, D), lambda r: (r, 0)),
                  pl.BlockSpec((D, P), lambda r: (0, 0))],
        out_specs=pl.BlockSpec((prb, P), lambda r: (r, 0)),
        out_shape=jax.ShapeDtypeStruct((2 * S, P), jnp.bfloat16),
    )(pe_pad, pos_to_pk.T.astype(jnp.bfloat16))

    bq = min(256, S)
    nq = S // bq
    wo = out_proj.reshape(EP, D).astype(jnp.bfloat16)
    out = pl.pallas_call(
        functools.partial(_attn_kernel, nq=nq, bq=bq, seq=S, e=E, p=P),
        grid=(B, nq),
        in_specs=[
            pl.BlockSpec((1, bq, EP), lambda b, i: (b, i, 0)),
            pl.BlockSpec((1, bq, E), lambda b, i: (b, i, 0)),
            pl.BlockSpec((1, S, P), lambda b, i: (b, 0, 0)),
            pl.BlockSpec((1, S, P), lambda b, i: (b, 0, 0)),
            pl.BlockSpec((2 * S, P), lambda b, i: (0, 0)),
            pl.BlockSpec((EP, D), lambda b, i: (0, 0)),
        ],
        out_specs=pl.BlockSpec((1, bq, D), lambda b, i: (b, i, 0)),
        out_shape=jax.ShapeDtypeStruct((B, S, D), jnp.float32),
        compiler_params=pltpu.CompilerParams(
            dimension_semantics=("parallel", "parallel")),
    )(q_all, sel, kk, vv, pos_k, wo)
    return out
